# baseline (speedup 1.0000x reference)
import functools

import jax
import jax.numpy as jnp
from jax import lax
from jax.experimental import pallas as pl
from jax.experimental.pallas import tpu as pltpu
from jax.experimental.pallas import tpu_sc as plsc

B = 16384
D = 64
DP = 128
NCOLS = 20
SCALE = 10.0

NW = 32
BPW = B // NW
CHUNK = 32
NCH = BPW // CHUNK
IPC = CHUNK * NCOLS
NIG = IPC // 128


@functools.partial(
    pl.kernel,
    out_type=jax.ShapeDtypeStruct((B * NCOLS,), jnp.float32),
    mesh=plsc.VectorSubcoreMesh(core_axis_name="c", subcore_axis_name="s"),
    scratch_types=[
        pltpu.VMEM((CHUNK,), jnp.int32),
        pltpu.VMEM((CHUNK + 16,), jnp.int32),
        pltpu.VMEM((IPC,), jnp.int32),
        pltpu.VMEM((IPC,), jnp.int32),
        pltpu.VMEM((CHUNK, DP), jnp.float32),
        pltpu.VMEM((IPC, DP), jnp.float32),
        pltpu.VMEM((IPC,), jnp.float32),
        pltpu.SemaphoreType.DMA,
        pltpu.SemaphoreType.DMA,
    ],
    compiler_params=pltpu.CompilerParams(needs_layout_passes=False),
)
def _mf_sc(urow_hbm, uoff_hbm, irow_hbm, ioff_hbm, user_hbm, item_hbm,
           out_hbm, urow_v, uoff_v, irow_v, ioff_v, urows_v, irows_v,
           scores_v, sem_u, sem_i):
    wid = lax.axis_index("s") * 2 + lax.axis_index("c")
    iota16 = lax.iota(jnp.int32, 16)

    def chunk_body(c, carry):
        base = wid * BPW + c * CHUNK
        pltpu.sync_copy(urow_hbm.at[pl.ds(base, CHUNK)], urow_v)
        pltpu.sync_copy(uoff_hbm.at[pl.ds(base, CHUNK)],
                        uoff_v.at[pl.ds(0, CHUNK)])
        pltpu.sync_copy(irow_hbm.at[pl.ds(base * NCOLS, IPC)], irow_v)
        pltpu.sync_copy(ioff_hbm.at[pl.ds(base * NCOLS, IPC)], ioff_v)
        cu = pltpu.async_copy(user_hbm.at[urow_v], urows_v, sem_u)
        cps = [
            pltpu.async_copy(item_hbm.at[irow_v.at[pl.ds(r * 128, 128)]],
                             irows_v.at[pl.ds(r * 128, 128)], sem_i)
            for r in range(NIG)
        ]
        cu.wait()
        for cp in cps:
            cp.wait()

        def bbody(b4, inner):
            res = [jnp.zeros((16,), jnp.float32) for _ in range(5)]
            uoffs = uoff_v[pl.ds(b4 * 4, 16)]
            ioffs = [ioff_v[pl.ds(b4 * 80 + v * 16, 16)] for v in range(5)]
            for bb in range(4):
                b = b4 * 4 + bb
                pu = uoffs[bb]
                u = [urows_v[b, pl.ds(pu + k * 16, 16)]
                     for k in range(D // 16)]
                for j in range(NCOLS):
                    row = b * NCOLS + j
                    o = bb * NCOLS + j
                    pi = ioffs[o // 16][o % 16]
                    prod = u[0] * irows_v[row, pl.ds(pi, 16)]
                    for k in range(1, D // 16):
                        prod = prod + u[k] * irows_v[row, pl.ds(pi + k * 16,
                                                                16)]
                    s = jnp.sum(prod)
                    res[o // 16] = jnp.where(iota16 == (o % 16), s,
                                             res[o // 16])
            for v in range(5):
                scores_v[pl.ds(b4 * 80 + v * 16, 16)] = res[v] * SCALE
            return inner

        lax.fori_loop(0, CHUNK // 4, bbody, 0)
        pltpu.sync_copy(scores_v, out_hbm.at[pl.ds(base * NCOLS, IPC)])
        return carry

    lax.fori_loop(0, NCH, chunk_body, 0)


def kernel(batch, user_emb, item_emb):
    b = batch.astype(jnp.int32)
    uidx = b[:, 0]
    iidx = b[:, 1:].reshape(B * NCOLS)
    user_p = user_emb.reshape(500000, DP)
    item_p = item_emb.reshape(500000, DP)
    out = _mf_sc(uidx >> 1, (uidx & 1) * D, iidx >> 1, (iidx & 1) * D,
                 user_p, item_p)
    return out.reshape(B, NCOLS)


# speedup vs baseline: 1.1658x; 1.1658x over previous
import functools

import jax
import jax.numpy as jnp
from jax import lax
from jax.experimental import pallas as pl
from jax.experimental.pallas import tpu as pltpu
from jax.experimental.pallas import tpu_sc as plsc

B = 16384
D = 64
DP = 128
NCOLS = 20
SCALE = 10.0

NW = 32
BPW = B // NW
CHUNK = 32
NCH = BPW // CHUNK
IPC = CHUNK * NCOLS
NIG = IPC // 128


@functools.partial(
    pl.kernel,
    out_type=jax.ShapeDtypeStruct((B * NCOLS,), jnp.float32),
    mesh=plsc.VectorSubcoreMesh(core_axis_name="c", subcore_axis_name="s"),
    scratch_types=[
        pltpu.VMEM((CHUNK + 16,), jnp.int32),
        pltpu.VMEM((IPC,), jnp.int32),
        pltpu.VMEM((IPC,), jnp.int32),
        pltpu.VMEM((CHUNK * 8, D), jnp.float32),
        pltpu.VMEM((IPC, DP), jnp.float32),
        pltpu.VMEM((IPC,), jnp.float32),
        pltpu.SemaphoreType.DMA,
        pltpu.SemaphoreType.DMA,
    ],
    compiler_params=pltpu.CompilerParams(needs_layout_passes=False),
)
def _mf_sc(uidx_hbm, irow_hbm, ioff_hbm, user_hbm, item_hbm,
           out_hbm, uidx_v, irow_v, ioff_v, ublk_v, irows_v,
           scores_v, sem_u, sem_i):
    wid = lax.axis_index("s") * 2 + lax.axis_index("c")
    iota16 = lax.iota(jnp.int32, 16)

    def chunk_body(c, carry):
        base = wid * BPW + c * CHUNK
        pltpu.sync_copy(uidx_hbm.at[pl.ds(base, CHUNK)],
                        uidx_v.at[pl.ds(0, CHUNK)])
        pltpu.sync_copy(irow_hbm.at[pl.ds(base * NCOLS, IPC)], irow_v)
        pltpu.sync_copy(ioff_hbm.at[pl.ds(base * NCOLS, IPC)], ioff_v)
        cps = [
            pltpu.async_copy(item_hbm.at[irow_v.at[pl.ds(r * 128, 128)]],
                             irows_v.at[pl.ds(r * 128, 128)], sem_i)
            for r in range(NIG)
        ]
        uvecs = [uidx_v[pl.ds(w * 16, 16)] for w in range(CHUNK // 16)]
        cus = []
        for b in range(CHUNK):
            u = uvecs[b // 16][b % 16]
            blk = pl.multiple_of((u >> 3) * 8, 8)
            cus.append(pltpu.async_copy(
                user_hbm.at[pl.ds(blk, 8)],
                ublk_v.at[pl.ds(b * 8, 8)], sem_u))
        for cp in cus:
            cp.wait()
        for cp in cps:
            cp.wait()

        def bbody(b4, inner):
            res = [jnp.zeros((16,), jnp.float32) for _ in range(5)]
            uvec = uidx_v[pl.ds(b4 * 4, 16)]
            ioffs = [ioff_v[pl.ds(b4 * 80 + v * 16, 16)] for v in range(5)]
            for bb in range(4):
                b = b4 * 4 + bb
                urow = b * 8 + (uvec[bb] & 7)
                u = [ublk_v[urow, pl.ds(k * 16, 16)]
                     for k in range(D // 16)]
                for j in range(NCOLS):
                    row = b * NCOLS + j
                    o = bb * NCOLS + j
                    pi = ioffs[o // 16][o % 16]
                    prod = u[0] * irows_v[row, pl.ds(pi, 16)]
                    for k in range(1, D // 16):
                        prod = prod + u[k] * irows_v[row, pl.ds(pi + k * 16,
                                                                16)]
                    s = jnp.sum(prod)
                    res[o // 16] = jnp.where(iota16 == (o % 16), s,
                                             res[o // 16])
            for v in range(5):
                scores_v[pl.ds(b4 * 80 + v * 16, 16)] = res[v] * SCALE
            return inner

        lax.fori_loop(0, CHUNK // 4, bbody, 0)
        pltpu.sync_copy(scores_v, out_hbm.at[pl.ds(base * NCOLS, IPC)])
        return carry

    lax.fori_loop(0, NCH, chunk_body, 0)


def kernel(batch, user_emb, item_emb):
    b = batch.astype(jnp.int32)
    uidx = b[:, 0]
    iidx = b[:, 1:].reshape(B * NCOLS)
    item_p = item_emb.reshape(500000, DP)
    out = _mf_sc(uidx, iidx >> 1, (iidx & 1) * D, user_emb, item_p)
    return out.reshape(B, NCOLS)


# speedup vs baseline: 1.2264x; 1.0520x over previous
import functools

import jax
import jax.numpy as jnp
from jax import lax
from jax.experimental import pallas as pl
from jax.experimental.pallas import tpu as pltpu
from jax.experimental.pallas import tpu_sc as plsc

B = 16384
D = 64
DP = 128
NCOLS = 20
SCALE = 10.0

NW = 32
BPW = B // NW
IPW = BPW * NCOLS
CHUNK = 8
NCH = BPW // CHUNK
IPC = CHUNK * NCOLS
IGS = (128, 32)


@functools.partial(
    pl.kernel,
    out_type=jax.ShapeDtypeStruct((B * NCOLS,), jnp.float32),
    mesh=plsc.VectorSubcoreMesh(core_axis_name="c", subcore_axis_name="s"),
    scratch_types=[
        pltpu.VMEM((BPW + 16,), jnp.int32),
        pltpu.VMEM((IPW,), jnp.int32),
        pltpu.VMEM((IPW,), jnp.int32),
        pltpu.VMEM((2, CHUNK * 8, D), jnp.float32),
        pltpu.VMEM((2, IPC, DP), jnp.float32),
        pltpu.VMEM((IPC,), jnp.float32),
        pltpu.SemaphoreType.DMA,
        pltpu.SemaphoreType.DMA,
    ],
    compiler_params=pltpu.CompilerParams(needs_layout_passes=False),
)
def _mf_sc(uidx_hbm, irow_hbm, ioff_hbm, user_hbm, item_hbm,
           out_hbm, uidx_v, irow_v, ioff_v, ublk_v, irows_v,
           scores_v, sem_u, sem_i):
    wid = lax.axis_index("s") * 2 + lax.axis_index("c")
    iota16 = lax.iota(jnp.int32, 16)
    wbase = wid * BPW

    pltpu.sync_copy(uidx_hbm.at[pl.ds(wbase, BPW)],
                    uidx_v.at[pl.ds(0, BPW)])
    pltpu.sync_copy(irow_hbm.at[pl.ds(wbase * NCOLS, IPW)], irow_v)
    pltpu.sync_copy(ioff_hbm.at[pl.ds(wbase * NCOLS, IPW)], ioff_v)

    def issue(c, buf):
        r0 = c * IPC
        o = 0
        for n in IGS:
            pltpu.async_copy(
                item_hbm.at[irow_v.at[pl.ds(r0 + o, n)]],
                irows_v.at[buf].at[pl.ds(o, n)], sem_i)
            o += n
        uvec = uidx_v[pl.ds(c * CHUNK, 16)]
        for b in range(CHUNK):
            u = uvec[b]
            blk = pl.multiple_of((u >> 3) * 8, 8)
            pltpu.async_copy(
                user_hbm.at[pl.ds(blk, 8)],
                ublk_v.at[buf].at[pl.ds(b * 8, 8)], sem_u)

    def drain(buf):
        o = 0
        for n in IGS:
            pltpu.make_async_copy(
                item_hbm.at[irow_v.at[pl.ds(0, n)]],
                irows_v.at[buf].at[pl.ds(o, n)], sem_i).wait()
            o += n
        for b in range(CHUNK):
            pltpu.make_async_copy(
                user_hbm.at[pl.ds(0, 8)],
                ublk_v.at[buf].at[pl.ds(b * 8, 8)], sem_u).wait()

    def compute(c, buf):
        def bbody(b4, inner):
            res = [jnp.zeros((16,), jnp.float32) for _ in range(5)]
            uv = uidx_v[pl.ds(c * CHUNK + b4 * 4, 16)]
            ioffs = [ioff_v[pl.ds(c * IPC + b4 * 80 + v * 16, 16)]
                     for v in range(5)]
            for bb in range(4):
                b = b4 * 4 + bb
                urow = b * 8 + (uv[bb] & 7)
                u = [ublk_v[buf, urow, pl.ds(k * 16, 16)]
                     for k in range(D // 16)]
                for j in range(NCOLS):
                    row = b * NCOLS + j
                    o = bb * NCOLS + j
                    pi = ioffs[o // 16][o % 16]
                    prod = u[0] * irows_v[buf, row, pl.ds(pi, 16)]
                    for k in range(1, D // 16):
                        prod = prod + u[k] * irows_v[buf, row,
                                                     pl.ds(pi + k * 16, 16)]
                    s = jnp.sum(prod)
                    res[o // 16] = jnp.where(iota16 == (o % 16), s,
                                             res[o // 16])
            for v in range(5):
                scores_v[pl.ds(b4 * 80 + v * 16, 16)] = res[v] * SCALE
            return inner

        lax.fori_loop(0, CHUNK // 4, bbody, 0)
        pltpu.sync_copy(scores_v,
                        out_hbm.at[pl.ds((wbase + c * CHUNK) * NCOLS, IPC)])

    issue(0, 0)

    def chunk_body(p, carry):
        c0 = p * 2
        issue(c0 + 1, 1)
        drain(0)
        compute(c0, 0)
        issue(c0 + 2, 0)
        drain(1)
        compute(c0 + 1, 1)
        return carry

    lax.fori_loop(0, NCH // 2 - 1, chunk_body, 0)
    issue(NCH - 1, 1)
    drain(0)
    compute(NCH - 2, 0)
    drain(1)
    compute(NCH - 1, 1)


def kernel(batch, user_emb, item_emb):
    b = batch.astype(jnp.int32)
    uidx = b[:, 0]
    iidx = b[:, 1:].reshape(B * NCOLS)
    item_p = item_emb.reshape(500000, DP)
    out = _mf_sc(uidx, iidx >> 1, (iidx & 1) * D, user_emb, item_p)
    return out.reshape(B, NCOLS)


# speedup vs baseline: 1.4636x; 1.1934x over previous
import functools

import jax
import jax.numpy as jnp
from jax import lax
from jax.experimental import pallas as pl
from jax.experimental.pallas import tpu as pltpu
from jax.experimental.pallas import tpu_sc as plsc

B = 16384
D = 64
DP = 128
NCOLS = 20
SCALE = 10.0

NW = 32
BPW = B // NW
IPW = BPW * NCOLS
CHUNK = 8
NCH = BPW // CHUNK
IPC = CHUNK * NCOLS
IGS = (128, 32)


@functools.partial(
    pl.kernel,
    out_type=jax.ShapeDtypeStruct((B * NCOLS,), jnp.float32),
    mesh=plsc.VectorSubcoreMesh(core_axis_name="c", subcore_axis_name="s"),
    scratch_types=[
        pltpu.VMEM((BPW + 16,), jnp.int32),
        pltpu.VMEM((IPW,), jnp.int32),
        pltpu.VMEM((IPW,), jnp.int32),
        pltpu.VMEM((2, CHUNK * 8, D), jnp.float32),
        pltpu.VMEM((2, IPC, DP), jnp.float32),
        pltpu.VMEM((IPC,), jnp.float32),
        pltpu.SemaphoreType.DMA,
        pltpu.SemaphoreType.DMA,
    ],
    compiler_params=pltpu.CompilerParams(needs_layout_passes=False),
)
def _mf_sc(uidx_hbm, irow_hbm, ioff_hbm, user_hbm, item_hbm,
           out_hbm, uidx_v, irow_v, ioff_v, ublk_v, irows_v,
           scores_v, sem_u, sem_i):
    wid = lax.axis_index("s") * 2 + lax.axis_index("c")
    iota16 = lax.iota(jnp.int32, 16)
    wbase = wid * BPW

    pltpu.sync_copy(uidx_hbm.at[pl.ds(wbase, BPW)],
                    uidx_v.at[pl.ds(0, BPW)])
    pltpu.sync_copy(irow_hbm.at[pl.ds(wbase * NCOLS, IPW)], irow_v)
    pltpu.sync_copy(ioff_hbm.at[pl.ds(wbase * NCOLS, IPW)], ioff_v)

    def issue(c, buf):
        r0 = c * IPC
        o = 0
        for n in IGS:
            pltpu.async_copy(
                item_hbm.at[irow_v.at[pl.ds(r0 + o, n)]],
                irows_v.at[buf].at[pl.ds(o, n)], sem_i)
            o += n
        uvec = uidx_v[pl.ds(c * CHUNK, 16)]
        for b in range(CHUNK):
            pltpu.async_copy(
                user_hbm.at[uvec[b] >> 3],
                ublk_v.at[buf].at[pl.ds(b * 8, 8)], sem_u)

    def drain(buf):
        o = 0
        for n in IGS:
            pltpu.make_async_copy(
                item_hbm.at[irow_v.at[pl.ds(0, n)]],
                irows_v.at[buf].at[pl.ds(o, n)], sem_i).wait()
            o += n
        for b in range(CHUNK):
            pltpu.make_async_copy(
                user_hbm.at[0],
                ublk_v.at[buf].at[pl.ds(b * 8, 8)], sem_u).wait()

    def compute(c, buf):
        def bbody(b4, inner):
            res = [jnp.zeros((16,), jnp.float32) for _ in range(5)]
            uv = uidx_v[pl.ds(c * CHUNK + b4 * 4, 16)]
            ioffs = [ioff_v[pl.ds(c * IPC + b4 * 80 + v * 16, 16)]
                     for v in range(5)]
            for bb in range(4):
                b = b4 * 4 + bb
                urow = b * 8 + (uv[bb] & 7)
                u = [ublk_v[buf, urow, pl.ds(k * 16, 16)]
                     for k in range(D // 16)]
                for j in range(NCOLS):
                    row = b * NCOLS + j
                    o = bb * NCOLS + j
                    pi = ioffs[o // 16][o % 16]
                    prod = u[0] * irows_v[buf, row, pl.ds(pi, 16)]
                    for k in range(1, D // 16):
                        prod = prod + u[k] * irows_v[buf, row,
                                                     pl.ds(pi + k * 16, 16)]
                    s = jnp.sum(prod)
                    res[o // 16] = jnp.where(iota16 == (o % 16), s,
                                             res[o // 16])
            for v in range(5):
                scores_v[pl.ds(b4 * 80 + v * 16, 16)] = res[v] * SCALE
            return inner

        lax.fori_loop(0, CHUNK // 4, bbody, 0)
        pltpu.sync_copy(scores_v,
                        out_hbm.at[pl.ds((wbase + c * CHUNK) * NCOLS, IPC)])

    issue(0, 0)

    def chunk_body(p, carry):
        c0 = p * 2
        issue(c0 + 1, 1)
        drain(0)
        compute(c0, 0)
        issue(c0 + 2, 0)
        drain(1)
        compute(c0 + 1, 1)
        return carry

    lax.fori_loop(0, NCH // 2 - 1, chunk_body, 0)
    issue(NCH - 1, 1)
    drain(0)
    compute(NCH - 2, 0)
    drain(1)
    compute(NCH - 1, 1)


def kernel(batch, user_emb, item_emb):
    b = batch.astype(jnp.int32)
    uidx = b[:, 0]
    iidx = b[:, 1:].reshape(B * NCOLS)
    user_p = user_emb.reshape(125000, 8, D)
    item_p = item_emb.reshape(500000, DP)
    out = _mf_sc(uidx, iidx >> 1, (iidx & 1) * D, user_p, item_p)
    return out.reshape(B, NCOLS)
